# Initial kernel scaffold; baseline (speedup 1.0000x reference)
import jax
import jax.numpy as jnp
from jax.experimental import pallas as pl


def kernel(node_type, edge_index, distance, graph_ids, params):
    raise NotImplementedError("write your pallas kernel here")


# speedup vs baseline: 2.4323x; 2.4323x over previous
import functools

import jax
import jax.numpy as jnp
from jax import lax
from jax.experimental import pallas as pl
from jax.experimental.pallas import tpu as pltpu
from jax.experimental.pallas import tpu_sc as plsc

N = 50000
E = 800000
DIM = 64
N_TYPES = 100
N_GRAPHS = 2000
CUTOFF = 5.0
N_CENTERS = 5
N_CONV = 3

CH_N = 2000
CHE = 4000
IDX_W = 125
IDX_H = 2
K_EDGE = IDX_W * IDX_H
G_PAD = 2048
NT = 16
EPT = E // NT
CZ = 2000
LN2 = 0.6931471805599453

_INTERPRET = False


def _sp05(x):
    a = 0.5 * x
    return 2.0 * (jnp.maximum(a, 0.0) + jnp.log(1.0 + jnp.exp(-jnp.abs(a))))


def _embed_block(nt_ref, emb_ref, w1_ref, node_ref, nna_ref, nnb_ref):
    nt = nt_ref[0, 0, :]
    oh = (nt[:, None] == lax.broadcasted_iota(jnp.int32, (CH_N, 128), 1))
    node = jnp.dot(oh.astype(jnp.float32), emb_ref[...],
                   preferred_element_type=jnp.float32)
    node_ref[...] = node
    nn = jnp.dot(node, w1_ref[...], preferred_element_type=jnp.float32)
    nna_ref[...] = nn[:, :32]
    nnb_ref[...] = nn[:, 32:]


def _embed(node_type, emb_pad, w1):
    nt3 = node_type.reshape(N // CH_N, 1, CH_N)
    return pl.pallas_call(
        _embed_block,
        grid=(N // CH_N,),
        in_specs=[pl.BlockSpec((1, 1, CH_N), lambda i: (i, 0, 0)),
                  pl.BlockSpec((128, DIM), lambda i: (0, 0)),
                  pl.BlockSpec((DIM, DIM), lambda i: (0, 0))],
        out_specs=[pl.BlockSpec((CH_N, DIM), lambda i: (i, 0)),
                   pl.BlockSpec((CH_N, 32), lambda i: (i, 0)),
                   pl.BlockSpec((CH_N, 32), lambda i: (i, 0))],
        out_shape=[jax.ShapeDtypeStruct((N, DIM), jnp.float32),
                   jax.ShapeDtypeStruct((N, 32), jnp.float32),
                   jax.ShapeDtypeStruct((N, 32), jnp.float32)],
        interpret=_INTERPRET,
    )(nt3, emb_pad, w1)


def _filter_block(d_ref, w1s_ref, b1s_ref, w2s_ref, b2s_ref, *out_refs):
    d = d_ref[0, 0, :][:, None]
    gap = CUTOFF / (N_CENTERS - 1)
    w1s = w1s_ref[...]
    b1s = b1s_ref[...]
    rbf = [jnp.exp(-(d - k * gap) ** 2 * (1.0 / gap)) for k in range(N_CENTERS)]
    for i in range(N_CONV):
        t = b1s[i]
        for k in range(N_CENTERS):
            t = t + rbf[k] * w1s[i, k][None, :]
        h1 = _sp05(t)
        h = jnp.dot(h1, w2s_ref[i], preferred_element_type=jnp.float32) \
            + b2s_ref[i]
        out_refs[2 * i][...] = h[:, :32]
        out_refs[2 * i + 1][...] = h[:, 32:]


def _filters(distance, w1s, b1s, w2s, b2s):
    d3 = distance.reshape(E // CHE, 1, CHE)
    out_specs = [pl.BlockSpec((CHE, 32), lambda i: (i, 0))] * 6
    out_shape = [jax.ShapeDtypeStruct((E, 32), jnp.float32)] * 6
    return pl.pallas_call(
        _filter_block,
        grid=(E // CHE,),
        in_specs=[pl.BlockSpec((1, 1, CHE), lambda i: (i, 0, 0)),
                  pl.BlockSpec((N_CONV, N_CENTERS, DIM), lambda i: (0, 0, 0)),
                  pl.BlockSpec((N_CONV, 1, DIM), lambda i: (0, 0, 0)),
                  pl.BlockSpec((N_CONV, DIM, DIM), lambda i: (0, 0, 0)),
                  pl.BlockSpec((N_CONV, 1, DIM), lambda i: (0, 0, 0))],
        out_specs=out_specs,
        out_shape=out_shape,
        interpret=_INTERPRET,
    )(d3, w1s, b1s, w2s, b2s)


def _update_block(cfa_ref, cfb_ref, node_ref, w2_ref, b2_ref, w3_ref, b3_ref,
                  w1n_ref, nodeo_ref, nna_ref, nnb_ref):
    cf = jnp.concatenate([cfa_ref[...], cfb_ref[...]], axis=1)
    cf1 = jnp.dot(cf, w2_ref[...], preferred_element_type=jnp.float32) \
        + b2_ref[...]
    s = _sp05(cf1)
    nd = node_ref[...] + jnp.dot(s, w3_ref[...],
                                 preferred_element_type=jnp.float32) \
        + b3_ref[...]
    nodeo_ref[...] = nd
    nn = jnp.dot(nd, w1n_ref[...], preferred_element_type=jnp.float32)
    nna_ref[...] = nn[:, :32]
    nnb_ref[...] = nn[:, 32:]


def _update(cf_a, cf_b, node, w2, b2, w3, b3, w1n):
    return pl.pallas_call(
        _update_block,
        grid=(N // CH_N,),
        in_specs=[pl.BlockSpec((CH_N, 32), lambda i: (i, 0)),
                  pl.BlockSpec((CH_N, 32), lambda i: (i, 0)),
                  pl.BlockSpec((CH_N, DIM), lambda i: (i, 0)),
                  pl.BlockSpec((DIM, DIM), lambda i: (0, 0)),
                  pl.BlockSpec((1, DIM), lambda i: (0, 0)),
                  pl.BlockSpec((DIM, DIM), lambda i: (0, 0)),
                  pl.BlockSpec((1, DIM), lambda i: (0, 0)),
                  pl.BlockSpec((DIM, DIM), lambda i: (0, 0))],
        out_specs=[pl.BlockSpec((CH_N, DIM), lambda i: (i, 0)),
                   pl.BlockSpec((CH_N, 32), lambda i: (i, 0)),
                   pl.BlockSpec((CH_N, 32), lambda i: (i, 0))],
        out_shape=[jax.ShapeDtypeStruct((N, DIM), jnp.float32),
                   jax.ShapeDtypeStruct((N, 32), jnp.float32),
                   jax.ShapeDtypeStruct((N, 32), jnp.float32)],
        interpret=_INTERPRET,
    )(cf_a, cf_b, node, w2, b2, w3, b3, w1n)


def _head_block(cfa_ref, cfb_ref, node_ref, w2_ref, b2_ref, w3_ref, b3_ref,
                d1w_ref, d1b_ref, d2w_ref, d2b_ref, res_ref):
    cf = jnp.concatenate([cfa_ref[...], cfb_ref[...]], axis=1)
    cf1 = jnp.dot(cf, w2_ref[...], preferred_element_type=jnp.float32) \
        + b2_ref[...]
    s = _sp05(cf1)
    nd = node_ref[...] + jnp.dot(s, w3_ref[...],
                                 preferred_element_type=jnp.float32) \
        + b3_ref[...]
    atom = jnp.dot(nd, d1w_ref[...], preferred_element_type=jnp.float32) \
        + d1b_ref[...]
    a2 = jnp.maximum(atom, 0.0) + jnp.log(1.0 + jnp.exp(-jnp.abs(atom))) - LN2
    r = jnp.sum(a2 * d2w_ref[...], axis=1)[:, None] + d2b_ref[0, 0]
    lane = lax.broadcasted_iota(jnp.int32, (CH_N, 16), 1)
    res_ref[...] = jnp.where(lane == 0, r,
                             jnp.where(lane == 1, 1.0, 0.0))


def _head(cf_a, cf_b, node, w2, b2, w3, b3, d1w, d1b, d2w, d2b):
    return pl.pallas_call(
        _head_block,
        grid=(N // CH_N,),
        in_specs=[pl.BlockSpec((CH_N, 32), lambda i: (i, 0)),
                  pl.BlockSpec((CH_N, 32), lambda i: (i, 0)),
                  pl.BlockSpec((CH_N, DIM), lambda i: (i, 0)),
                  pl.BlockSpec((DIM, DIM), lambda i: (0, 0)),
                  pl.BlockSpec((1, DIM), lambda i: (0, 0)),
                  pl.BlockSpec((DIM, DIM), lambda i: (0, 0)),
                  pl.BlockSpec((1, DIM), lambda i: (0, 0)),
                  pl.BlockSpec((DIM, DIM), lambda i: (0, 0)),
                  pl.BlockSpec((1, DIM), lambda i: (0, 0)),
                  pl.BlockSpec((1, DIM), lambda i: (0, 0)),
                  pl.BlockSpec((1, 1), lambda i: (0, 0))],
        out_specs=pl.BlockSpec((CH_N, 16), lambda i: (i, 0)),
        out_shape=jax.ShapeDtypeStruct((N, 16), jnp.float32),
        interpret=_INTERPRET,
    )(cf_a, cf_b, node, w2, b2, w3, b3, d1w, d1b, d2w, d2b)


def _conv_sc(nn_a, nn_b, h_a, h_b, src2, dst2, zeros_nc):
    mesh = plsc.VectorSubcoreMesh(core_axis_name="c", subcore_axis_name="s")

    @functools.partial(
        pl.kernel,
        out_type=[jax.ShapeDtypeStruct((N, 32), jnp.float32),
                  jax.ShapeDtypeStruct((N, 32), jnp.float32)],
        mesh=mesh,
        scratch_types=[
            pltpu.VMEM((IDX_H, IDX_W), jnp.int32),
            pltpu.VMEM((IDX_H, IDX_W), jnp.int32),
            pltpu.VMEM((IDX_H, IDX_W, 32), jnp.float32),
            pltpu.VMEM((K_EDGE, 32), jnp.float32),
            pltpu.VMEM_SHARED((N, 32), jnp.float32),
            pltpu.SemaphoreType.DMA,
        ],
        compiler_params=pltpu.CompilerParams(use_tc_tiling_on_sc=False),
    )
    def k(nna_hbm, nnb_hbm, ha_hbm, hb_hbm, src_hbm, dst_hbm, z_hbm,
          outa_hbm, outb_hbm, sidx, didx, gv, hv, acc, sem):
        c = lax.axis_index("c")
        s = lax.axis_index("s")

        def run(nn_ref, h_ref, out_ref):
            def zchunk(j, _):
                ci = s + j * NT

                @pl.when(ci < N // CZ)
                def _():
                    pltpu.sync_copy(z_hbm.at[pl.ds(ci * CZ, CZ)],
                                    acc.at[pl.ds(ci * CZ, CZ)])

                return 0

            lax.fori_loop(0, (N // CZ + NT - 1) // NT, zchunk, 0)
            plsc.subcore_barrier()

            def chunk(i, _):
                rb = s * (EPT // IDX_W) + i * IDX_H
                pltpu.sync_copy(src_hbm.at[pl.ds(rb, IDX_H)], sidx)
                pltpu.sync_copy(dst_hbm.at[pl.ds(rb, IDX_H)], didx)
                cps = [pltpu.async_copy(nn_ref.at[sidx.at[a]], gv.at[a], sem)
                       for a in range(IDX_H)]
                eb = s * EPT + i * K_EDGE
                pltpu.sync_copy(h_ref.at[pl.ds(eb, K_EDGE)], hv)
                for cp in cps:
                    cp.wait()

                def mulrow(r, _):
                    for a in range(IDX_H):
                        for t in range(2):
                            sl = pl.ds(t * 16, 16)
                            gv[a, r, sl] = gv[a, r, sl] * hv[a * IDX_W + r, sl]
                    return 0

                lax.fori_loop(0, IDX_W, mulrow, 0)
                for a in range(IDX_H):
                    pltpu.sync_copy(gv.at[a], acc.at[didx.at[a]], add=True)
                return 0

            lax.fori_loop(0, EPT // K_EDGE, chunk, 0)
            plsc.subcore_barrier()

            def ochunk(j, _):
                ci = s + j * NT

                @pl.when(ci < N // CZ)
                def _():
                    pltpu.sync_copy(acc.at[pl.ds(ci * CZ, CZ)],
                                    out_ref.at[pl.ds(ci * CZ, CZ)])

                return 0

            lax.fori_loop(0, (N // CZ + NT - 1) // NT, ochunk, 0)

        @pl.when(c == 0)
        def _():
            run(nna_hbm, ha_hbm, outa_hbm)

        @pl.when(c == 1)
        def _():
            run(nnb_hbm, hb_hbm, outb_hbm)

    return k(nn_a, nn_b, h_a, h_b, src2, dst2, zeros_nc)


def _pool_sc(res3, gid2, zeros_pool):
    mesh = plsc.VectorSubcoreMesh(core_axis_name="c", subcore_axis_name="s")
    nchunk = N // K_EDGE
    iters = (nchunk + NT - 1) // NT
    gpt = G_PAD // NT

    @functools.partial(
        pl.kernel,
        out_type=jax.ShapeDtypeStruct((G_PAD, 16), jnp.float32),
        mesh=mesh,
        scratch_types=[
            pltpu.VMEM((IDX_H, IDX_W), jnp.int32),
            pltpu.VMEM((IDX_H, IDX_W, 16), jnp.float32),
            pltpu.VMEM_SHARED((G_PAD, 16), jnp.float32),
        ],
        compiler_params=pltpu.CompilerParams(use_tc_tiling_on_sc=False),
    )
    def k(res_hbm, gid_hbm, z_hbm, out_hbm, gidx, vv, acc):
        c = lax.axis_index("c")
        s = lax.axis_index("s")

        @pl.when(c == 0)
        def _():
            pltpu.sync_copy(z_hbm.at[pl.ds(s * gpt, gpt)],
                            acc.at[pl.ds(s * gpt, gpt)])
            plsc.subcore_barrier()

            def it(j, _):
                ci = s + j * NT

                @pl.when(ci < nchunk)
                def _():
                    rb = ci * IDX_H
                    pltpu.sync_copy(gid_hbm.at[pl.ds(rb, IDX_H)], gidx)
                    pltpu.sync_copy(res_hbm.at[pl.ds(rb, IDX_H)], vv)
                    for a in range(IDX_H):
                        pltpu.sync_copy(vv.at[a], acc.at[gidx.at[a]],
                                        add=True)

                return 0

            lax.fori_loop(0, iters, it, 0)
            plsc.subcore_barrier()
            pltpu.sync_copy(acc.at[pl.ds(s * gpt, gpt)],
                            out_hbm.at[pl.ds(s * gpt, gpt)])

    return k(res3, gid2, zeros_pool)


def kernel(node_type, edge_index, distance, graph_ids, params):
    p = params
    src2 = edge_index[0].reshape(E // IDX_W, IDX_W)
    dst2 = edge_index[1].reshape(E // IDX_W, IDX_W)
    emb_pad = jnp.zeros((128, DIM), jnp.float32).at[:N_TYPES].set(p["emb"])
    w1s = jnp.stack([p["conv%d" % i]["cf_W1"] for i in range(N_CONV)])
    b1s = jnp.stack([p["conv%d" % i]["cf_b1"].reshape(1, DIM)
                     for i in range(N_CONV)])
    w2s = jnp.stack([p["conv%d" % i]["cf_W2"] for i in range(N_CONV)])
    b2s = jnp.stack([p["conv%d" % i]["cf_b2"].reshape(1, DIM)
                     for i in range(N_CONV)])
    hs = _filters(distance.reshape(E), w1s, b1s, w2s, b2s)
    node, nn_a, nn_b = _embed(node_type, emb_pad, p["conv0"]["W1"])
    zeros_nc = jnp.zeros((N, 32), jnp.float32)
    res = None
    for i in range(N_CONV):
        cf_a, cf_b = _conv_sc(nn_a, nn_b, hs[2 * i], hs[2 * i + 1],
                              src2, dst2, zeros_nc)
        ci = p["conv%d" % i]
        if i < N_CONV - 1:
            w1n = p["conv%d" % (i + 1)]["W1"]
            node, nn_a, nn_b = _update(
                cf_a, cf_b, node, ci["W2"], ci["b2"].reshape(1, DIM),
                ci["W3"], ci["b3"].reshape(1, DIM), w1n)
        else:
            res = _head(
                cf_a, cf_b, node, ci["W2"], ci["b2"].reshape(1, DIM),
                ci["W3"], ci["b3"].reshape(1, DIM),
                p["d1_W"], p["d1_b"].reshape(1, DIM),
                p["d2_W"].reshape(1, DIM), p["d2_b"].reshape(1, 1))
    res3 = res.reshape(N // IDX_W, IDX_W, 16)
    gid2 = graph_ids.reshape(N // IDX_W, IDX_W)
    pooled = _pool_sc(res3, gid2, jnp.zeros((G_PAD, 16), jnp.float32))
    return pooled[:N_GRAPHS, 0:1] / jnp.maximum(pooled[:N_GRAPHS, 1:2], 1.0)
